# baseline (speedup 1.0000x reference)
import jax
import jax.numpy as jnp
from jax.experimental import pallas as pl
from jax.experimental.pallas import tpu as pltpu

_GRID = 16
_BLK_C = 65536 // _GRID


def _nan_free_body(x_ref, out_ref):
    i = pl.program_id(0)
    blk = x_ref[...]
    ok = jnp.where(jnp.any(blk != blk), 0, 1).astype(jnp.int32)

    @pl.when(i == 0)
    def _init():
        out_ref[0, 0] = ok

    @pl.when(i > 0)
    def _acc():
        out_ref[0, 0] = jnp.minimum(out_ref[0, 0], ok)


@jax.jit
def kernel(x):
    ok = pl.pallas_call(
        _nan_free_body,
        grid=(_GRID,),
        in_specs=[pl.BlockSpec((8, 32, _BLK_C), lambda i: (0, 0, i))],
        out_specs=pl.BlockSpec(
            block_shape=(1, 1),
            index_map=lambda i: (0, 0),
            memory_space=pltpu.SMEM,
        ),
        out_shape=jax.ShapeDtypeStruct((1, 1), jnp.int32),
    )(x)
    return ok[0, 0].astype(jnp.bool_)


# speedup vs baseline: 1.1393x; 1.1393x over previous
import jax
import jax.numpy as jnp
from jax.experimental import pallas as pl
from jax.experimental.pallas import tpu as pltpu

_GRID = 4
_BLK_C = 65536 // _GRID


def _nan_free_body(x_ref, out_ref):
    i = pl.program_id(0)
    blk = x_ref[...]
    ok = jnp.where(jnp.any(blk != blk), 0, 1).astype(jnp.int32)

    @pl.when(i == 0)
    def _init():
        out_ref[0, 0] = ok

    @pl.when(i > 0)
    def _acc():
        out_ref[0, 0] = jnp.minimum(out_ref[0, 0], ok)


@jax.jit
def kernel(x):
    ok = pl.pallas_call(
        _nan_free_body,
        grid=(_GRID,),
        in_specs=[pl.BlockSpec((8, 32, _BLK_C), lambda i: (0, 0, i))],
        out_specs=pl.BlockSpec(
            block_shape=(1, 1),
            index_map=lambda i: (0, 0),
            memory_space=pltpu.SMEM,
        ),
        out_shape=jax.ShapeDtypeStruct((1, 1), jnp.int32),
    )(x)
    return ok[0, 0].astype(jnp.bool_)


# speedup vs baseline: 1.2299x; 1.0795x over previous
import jax
import jax.numpy as jnp
from jax.experimental import pallas as pl
from jax.experimental.pallas import tpu as pltpu

_GRID = 4
_BLK_C = 65536 // _GRID
_MAG_MASK = 0x7FFFFFFF
_INF_BITS = 0x7F800000


def _nan_free_body(x_ref, out_ref, acc_ref):
    i = pl.program_id(0)
    bits = jax.lax.bitcast_convert_type(x_ref[...], jnp.int32)
    m = jnp.max(bits & _MAG_MASK)

    @pl.when(i == 0)
    def _init():
        acc_ref[0] = m

    @pl.when(i > 0)
    def _acc():
        acc_ref[0] = jnp.maximum(acc_ref[0], m)

    @pl.when(i == _GRID - 1)
    def _finalize():
        out_ref[0, 0] = jnp.where(acc_ref[0] <= _INF_BITS, 1, 0).astype(jnp.int32)


@jax.jit
def kernel(x):
    ok = pl.pallas_call(
        _nan_free_body,
        grid=(_GRID,),
        in_specs=[pl.BlockSpec((8, 32, _BLK_C), lambda i: (0, 0, i))],
        out_specs=pl.BlockSpec(
            block_shape=(1, 1),
            index_map=lambda i: (0, 0),
            memory_space=pltpu.SMEM,
        ),
        out_shape=jax.ShapeDtypeStruct((1, 1), jnp.int32),
        scratch_shapes=[pltpu.SMEM((1,), jnp.int32)],
    )(x)
    return ok[0, 0].astype(jnp.bool_)


# speedup vs baseline: 1.2465x; 1.0135x over previous
import jax
import jax.numpy as jnp
from jax.experimental import pallas as pl
from jax.experimental.pallas import tpu as pltpu

_GRID = 8
_BLK_C = 65536 // _GRID
_MAG_MASK = 0x7FFFFFFF
_INF_BITS = 0x7F800000


def _nan_free_body(x_ref, out_ref, acc_ref):
    i = pl.program_id(0)
    bits = jax.lax.bitcast_convert_type(x_ref[...], jnp.int32)
    m = jnp.max(bits & _MAG_MASK)

    @pl.when(i == 0)
    def _init():
        acc_ref[0] = m

    @pl.when(i > 0)
    def _acc():
        acc_ref[0] = jnp.maximum(acc_ref[0], m)

    @pl.when(i == _GRID - 1)
    def _finalize():
        out_ref[0, 0] = jnp.where(acc_ref[0] <= _INF_BITS, 1, 0).astype(jnp.int32)


@jax.jit
def kernel(x):
    ok = pl.pallas_call(
        _nan_free_body,
        grid=(_GRID,),
        in_specs=[pl.BlockSpec((8, 32, _BLK_C), lambda i: (0, 0, i))],
        out_specs=pl.BlockSpec(
            block_shape=(1, 1),
            index_map=lambda i: (0, 0),
            memory_space=pltpu.SMEM,
        ),
        out_shape=jax.ShapeDtypeStruct((1, 1), jnp.int32),
        scratch_shapes=[pltpu.SMEM((1,), jnp.int32)],
    )(x)
    return ok[0, 0].astype(jnp.bool_)


# speedup vs baseline: 1.2611x; 1.0117x over previous
import jax
import jax.numpy as jnp
from jax.experimental import pallas as pl
from jax.experimental.pallas import tpu as pltpu

_GRID = 8
_BLK_C = 65536
_MAG_MASK = 0x7FFFFFFF
_INF_BITS = 0x7F800000


def _nan_free_body(x_ref, out_ref, acc_ref):
    i = pl.program_id(0)
    bits = jax.lax.bitcast_convert_type(x_ref[...], jnp.int32)
    m = jnp.max(bits & _MAG_MASK)

    @pl.when(i == 0)
    def _init():
        acc_ref[0] = m

    @pl.when(i > 0)
    def _acc():
        acc_ref[0] = jnp.maximum(acc_ref[0], m)

    @pl.when(i == _GRID - 1)
    def _finalize():
        out_ref[0, 0] = jnp.where(acc_ref[0] <= _INF_BITS, 1, 0).astype(jnp.int32)


@jax.jit
def kernel(x):
    ok = pl.pallas_call(
        _nan_free_body,
        grid=(_GRID,),
        in_specs=[pl.BlockSpec((1, 32, _BLK_C), lambda i: (i, 0, 0))],
        out_specs=pl.BlockSpec(
            block_shape=(1, 1),
            index_map=lambda i: (0, 0),
            memory_space=pltpu.SMEM,
        ),
        out_shape=jax.ShapeDtypeStruct((1, 1), jnp.int32),
        scratch_shapes=[pltpu.SMEM((1,), jnp.int32)],
    )(x)
    return ok[0, 0].astype(jnp.bool_)
